# baseline (speedup 1.0000x reference)
import functools

import jax
import jax.numpy as jnp
from jax import lax
from jax.experimental import pallas as pl
from jax.experimental.pallas import tpu as pltpu
from jax.experimental.pallas import tpu_sc as plsc

_V = 100000
_B = 1024
_EPS = 0.1 / (_V - 2)
_CONF = 0.9
_NW = 32
_RPW = _B // _NW
_L = 16
_G = _RPW // 8
_W = 8192
_VT = 99968
_NFULL = 11


def _lanes():
    return lax.broadcasted_iota(jnp.int32, (_L,), 0)


def _fill_buf(ref, rows, width, values):
    vecs = [jnp.full((_L,), v, dtype=jnp.float32) for v in values]

    def body(c, carry):
        base = c * _L
        for i in range(rows):
            ref[i, pl.ds(base, _L)] = vecs[i]
        return carry

    lax.fori_loop(0, width // _L, body, 0)


def _group_dmas(out_hbm, r0, buf_a, buf_b, sem):
    copies = [
        pltpu.make_async_copy(buf_b, out_hbm.at[pl.ds(r0, 8), pl.ds(0, 128)], sem),
        pltpu.make_async_copy(
            buf_a.at[:, pl.ds(0, _W - 128)],
            out_hbm.at[pl.ds(r0, 8), pl.ds(128, _W - 128)], sem),
    ]
    for s in range(1, 1 + _NFULL):
        copies.append(pltpu.make_async_copy(
            buf_a, out_hbm.at[pl.ds(r0, 8), pl.ds(s * _W, _W)], sem))
    copies.append(pltpu.make_async_copy(
        buf_a.at[:, pl.ds(0, _VT - 12 * _W)],
        out_hbm.at[pl.ds(r0, 8), pl.ds(12 * _W, _VT - 12 * _W)], sem))
    return copies


def _last_tile_dma(out_hbm, r0, buf_a, sem, t0):
    a = pl.multiple_of(jnp.bitwise_and(_VT + 0 * t0, -128), 128)
    return pltpu.make_async_copy(
        buf_a.at[:, pl.ds(0, 128)], out_hbm.at[pl.ds(r0, 8), pl.ds(a, 128)], sem)


def _sc_body(trg_hbm, out_hbm, buf_a, buf_b, patch, trg_v, sem_g, sem_p):
    wid = lax.axis_index("s") * 2 + lax.axis_index("c")
    base = wid * _RPW

    pltpu.sync_copy(trg_hbm.at[pl.ds(base, _RPW)], trg_v)

    lanes = _lanes()
    ts = []
    for c in range(_RPW // _L):
        vec = trg_v[pl.ds(c * _L, _L)]
        ts.extend(vec[l] for l in range(_L))

    grp_ok = []
    for g in range(_G):
        n_pad = functools.reduce(
            lambda x, y: x + y,
            [jnp.where(ts[g * 8 + i] == 0, 1, 0) for i in range(8)])
        grp_ok.append(n_pad == 0)

    _fill_buf(buf_a, 8, _W, [_EPS] * 8)
    eps_vec = jnp.full((_L,), _EPS, dtype=jnp.float32)
    v0 = jnp.where(lanes == 0, 0.0, _EPS).astype(jnp.float32)
    for i in range(8):
        buf_b[i, pl.ds(0, _L)] = v0
        for c in range(1, 128 // _L):
            buf_b[i, pl.ds(c * _L, _L)] = eps_vec

    def row0(g):
        return pl.multiple_of(base + g * 8, 8)

    for g in range(_G):
        @pl.when(grp_ok[g])
        def _(g=g):
            for cp in _group_dmas(out_hbm, row0(g), buf_a, buf_b, sem_g):
                cp.start()
            _last_tile_dma(out_hbm, row0(g), buf_a, sem_g, ts[0]).start()
        if g >= 1:
            @pl.when(grp_ok[g - 1])
            def _(g=g):
                for cp in _group_dmas(out_hbm, row0(g - 1), buf_a, buf_b, sem_g):
                    cp.wait()
                _last_tile_dma(out_hbm, row0(g - 1), buf_a, sem_g, ts[0]).wait()
    @pl.when(grp_ok[_G - 1])
    def _():
        for cp in _group_dmas(out_hbm, row0(_G - 1), buf_a, buf_b, sem_g):
            cp.wait()
        _last_tile_dma(out_hbm, row0(_G - 1), buf_a, sem_g, ts[0]).wait()

    for g in range(_G):
        @pl.when(jnp.logical_not(grp_ok[g]))
        def _(g=g):
            rvals = [jnp.where(ts[g * 8 + i] == 0, 0.0, _EPS) for i in range(8)]
            rvecs = [jnp.full((_L,), 1.0, jnp.float32) * v for v in rvals]

            def body(c, carry):
                bb = c * _L
                for i in range(8):
                    buf_a[i, pl.ds(bb, _L)] = rvecs[i]
                return carry

            lax.fori_loop(0, _W // _L, body, 0)
            for i in range(8):
                buf_b[i, pl.ds(0, _L)] = jnp.where(lanes == 0, 0.0, rvals[i])
                for c in range(1, 128 // _L):
                    buf_b[i, pl.ds(c * _L, _L)] = rvecs[i]
            for cp in _group_dmas(out_hbm, row0(g), buf_a, buf_b, sem_g):
                cp.start()
            _last_tile_dma(out_hbm, row0(g), buf_a, sem_g, ts[0]).start()
            for cp in _group_dmas(out_hbm, row0(g), buf_a, buf_b, sem_g):
                cp.wait()
            _last_tile_dma(out_hbm, row0(g), buf_a, sem_g, ts[0]).wait()

    for r in range(_RPW):
        @pl.when(ts[r] != 0)
        def _(r=r):
            g = r // 8
            a = pl.multiple_of(jnp.bitwise_and(ts[r], -128), 128)

            def body(c, carry):
                cols = a + c * _L + lanes
                for i in range(8):
                    ti = ts[g * 8 + i]
                    v = jnp.where(cols == ti, _CONF, _EPS).astype(jnp.float32)
                    v = jnp.where(cols == 0, 0.0, v)
                    v = jnp.where(ti == 0, 0.0, v)
                    patch[r, i, pl.ds(c * _L, _L)] = v
                return carry

            lax.fori_loop(0, 128 // _L, body, 0)
            pltpu.make_async_copy(
                patch.at[r],
                out_hbm.at[pl.ds(row0(g), 8), pl.ds(a, 128)], sem_p).start()
    for r in range(_RPW):
        @pl.when(ts[r] != 0)
        def _(r=r):
            a = pl.multiple_of(jnp.bitwise_and(ts[r], -128), 128)
            pltpu.make_async_copy(
                patch.at[r],
                out_hbm.at[pl.ds(row0(r // 8), 8), pl.ds(a, 128)], sem_p).wait()


def kernel(trg_token_ids_batch):
    trg = trg_token_ids_batch.reshape(_B)
    run = functools.partial(
        pl.kernel,
        out_type=jax.ShapeDtypeStruct((_B, _V), jnp.float32),
        mesh=plsc.VectorSubcoreMesh(core_axis_name="c", subcore_axis_name="s"),
        scratch_types=[
            pltpu.VMEM((8, _W), jnp.float32),
            pltpu.VMEM((8, 128), jnp.float32),
            pltpu.VMEM((_RPW, 8, 128), jnp.float32),
            pltpu.VMEM((_RPW,), jnp.int32),
            pltpu.SemaphoreType.DMA,
            pltpu.SemaphoreType.DMA,
        ],
    )(_sc_body)
    return run(trg)


# speedup vs baseline: 3.3655x; 3.3655x over previous
import functools

import jax
import jax.numpy as jnp
from jax import lax
from jax.experimental import pallas as pl
from jax.experimental.pallas import tpu as pltpu
from jax.experimental.pallas import tpu_sc as plsc

_V = 100000
_B = 1024
_EPS = 0.1 / (_V - 2)
_CONF = 0.9
_NW = 32
_BPW = _B // _NW
_L = 16
_BR = 64
_NUNITS = _V // _BR
_TAIL = _V - _NUNITS * _BR
_UPW = (_NUNITS + 1 + _NW - 1) // _NW


def _lanes():
    return lax.broadcasted_iota(jnp.int32, (_L,), 0)


def _unit_copies(out_hbm, u, tmpl, first, sem):
    off = pl.multiple_of(u * _BR, _BR)
    c_first = [
        pltpu.make_async_copy(first, out_hbm.at[pl.ds(0, 8)], sem),
        pltpu.make_async_copy(
            tmpl.at[pl.ds(0, _BR - 8)], out_hbm.at[pl.ds(8, _BR - 8)], sem),
    ]
    c_mid = [pltpu.make_async_copy(tmpl, out_hbm.at[pl.ds(off, _BR)], sem)]
    c_tail = [pltpu.make_async_copy(
        tmpl.at[pl.ds(0, _TAIL)],
        out_hbm.at[pl.ds(pl.multiple_of(_NUNITS * _BR + 0 * u, 8), _TAIL)],
        sem)]
    return c_first, c_mid, c_tail


def _for_units(wid, out_hbm, tmpl, first, sem, action):
    def body(k, carry):
        u = wid + k * _NW
        c_first, c_mid, c_tail = _unit_copies(out_hbm, u, tmpl, first, sem)

        @pl.when(u == 0)
        def _():
            for cp in c_first:
                action(cp)

        @pl.when((u > 0) & (u < _NUNITS))
        def _():
            for cp in c_mid:
                action(cp)

        @pl.when(u == _NUNITS)
        def _():
            for cp in c_tail:
                action(cp)

        return carry

    lax.fori_loop(0, _UPW, body, 0)


def _sc_body(trg_hbm, out_hbm, tmpl, first, patch, trg_v, sem_u, sem_p):
    wid = lax.axis_index("s") * 2 + lax.axis_index("c")
    base = wid * _BPW

    pltpu.sync_copy(trg_hbm, trg_v)

    lanes = _lanes()
    zeros = jnp.zeros((_L,), jnp.float32)

    def tbody(c, carry):
        cb = pl.multiple_of(c * _L, _L)
        tv = trg_v[pl.ds(cb, _L)]
        vec = jnp.where(tv == 0, 0.0, _EPS).astype(jnp.float32)
        for i in range(_BR):
            tmpl[i, pl.ds(cb, _L)] = vec
        first[0, pl.ds(cb, _L)] = zeros
        for i in range(1, 8):
            first[i, pl.ds(cb, _L)] = vec
        return carry

    lax.fori_loop(0, _B // _L, tbody, 0)

    _for_units(wid, out_hbm, tmpl, first, sem_u, lambda cp: cp.start())
    _for_units(wid, out_hbm, tmpl, first, sem_u, lambda cp: cp.wait())

    ts = []
    for c in range(_BPW // _L):
        vec = trg_v[pl.ds(pl.multiple_of(base + c * _L, _L), _L)]
        ts.extend(vec[l] for l in range(_L))

    def patch_dma(r, t):
        v8 = pl.multiple_of(jnp.bitwise_and(t, -8), 8)
        bcol = pl.multiple_of(jnp.bitwise_and(base + r, -128), 128)
        return pltpu.make_async_copy(
            patch.at[r],
            out_hbm.at[pl.ds(v8, 8), pl.ds(bcol, 128)], sem_p)

    for r in range(_BPW):
        @pl.when(ts[r] != 0)
        def _(r=r):
            t = ts[r]
            v8 = jnp.bitwise_and(t, -8)
            bcol = jnp.bitwise_and(base + r, -128)

            def pbody(c, carry):
                tw = trg_v[pl.ds(bcol + c * _L, _L)]
                colpad = tw == 0
                for i in range(8):
                    v = v8 + i
                    val = jnp.where(tw == v, _CONF, _EPS).astype(jnp.float32)
                    val = jnp.where(colpad, 0.0, val)
                    val = jnp.where(v == 0, 0.0, val)
                    patch[r, i, pl.ds(c * _L, _L)] = val
                return carry

            lax.fori_loop(0, 128 // _L, pbody, 0)
            patch_dma(r, t).start()
    for r in range(_BPW):
        @pl.when(ts[r] != 0)
        def _(r=r):
            patch_dma(r, ts[r]).wait()


def kernel(trg_token_ids_batch):
    trg = trg_token_ids_batch.reshape(_B)
    run = functools.partial(
        pl.kernel,
        out_type=jax.ShapeDtypeStruct((_V, _B), jnp.float32),
        mesh=plsc.VectorSubcoreMesh(core_axis_name="c", subcore_axis_name="s"),
        scratch_types=[
            pltpu.VMEM((_BR, _B), jnp.float32),
            pltpu.VMEM((8, _B), jnp.float32),
            pltpu.VMEM((_BPW, 8, 128), jnp.float32),
            pltpu.VMEM((_B,), jnp.int32),
            pltpu.SemaphoreType.DMA,
            pltpu.SemaphoreType.DMA,
        ],
    )(_sc_body)
    return run(trg).T
